# baseline (speedup 1.0000x reference)
import functools

import jax
import jax.numpy as jnp
from jax import lax
from jax.experimental import pallas as pl
from jax.experimental.pallas import tpu as pltpu
from jax.experimental.pallas import tpu_sc as plsc

N_CAT = 26
VP1 = 100001
EMB = 32
BATCH = 16384
RWS = N_CAT * EMB

_NUM_CORES = 2
_NUM_SUBCORES = 16
_FPC = N_CAT // _NUM_CORES


_CH = 8192
_NCH = (VP1 + _CH - 1) // _CH


def _max_body(tab_ref, out_ref):
    k = pl.program_id(1)
    blk = jnp.abs(tab_ref[...])
    col = lax.broadcasted_iota(jnp.int32, blk.shape, 1) + k * _CH
    blk = jnp.where(col < VP1, blk, 0.0)
    m = jnp.max(blk)

    @pl.when(k == 0)
    def _():
        out_ref[0, 0, :] = jnp.full((128,), m)

    @pl.when(k > 0)
    def _():
        out_ref[0, 0, :] = jnp.maximum(out_ref[0, 0, :], m)


def _field_maxes(tab_t):
    return pl.pallas_call(
        _max_body,
        grid=(N_CAT, _NCH),
        in_specs=[pl.BlockSpec((EMB, _CH), lambda f, k: (f, k))],
        out_specs=pl.BlockSpec((1, 1, 128), lambda f, k: (f, 0, 0)),
        out_shape=jax.ShapeDtypeStruct((N_CAT, 1, 128), jnp.float32),
    )(tab_t)


_HB = 8192


def _sc_gather(x_t, tab_t):
    mesh = plsc.VectorSubcoreMesh(core_axis_name="c", subcore_axis_name="s")

    @functools.partial(
        pl.kernel,
        mesh=mesh,
        compiler_params=pltpu.CompilerParams(
            use_tc_tiling_on_sc=True, needs_layout_passes=False
        ),
        out_type=jax.ShapeDtypeStruct((RWS, BATCH), jnp.float32),
        scratch_types=[
            pltpu.VMEM((VP1,), jnp.float32),
            pltpu.VMEM((_HB,), jnp.int32),
            pltpu.VMEM((_HB,), jnp.float32),
        ],
    )
    def k(x_hbm, tab_hbm, raw_hbm, tvec_v, xcol_v, obuf_v):
        c = lax.axis_index("c")
        s = lax.axis_index("s")

        def field_body(tf, carry):
            f = c * _FPC + tf

            def j_body(jj, carry):
                r = f * EMB + s + _NUM_SUBCORES * jj
                pltpu.sync_copy(tab_hbm.at[r], tvec_v)

                def b_body(bc, carry):
                    b0 = bc * _HB
                    pltpu.sync_copy(x_hbm.at[f, pl.ds(b0, _HB)], xcol_v)

                    def v_body(v, carry):
                        iv = xcol_v[pl.ds(v * 16, 16)] + 1
                        obuf_v[pl.ds(v * 16, 16)] = plsc.load_gather(
                            tvec_v, [iv]
                        )
                        return carry

                    lax.fori_loop(0, _HB // 16, v_body, 0)
                    pltpu.sync_copy(obuf_v, raw_hbm.at[r, pl.ds(b0, _HB)])
                    return carry

                return lax.fori_loop(0, BATCH // _HB, b_body, carry)

            return lax.fori_loop(0, 2, j_body, carry)

        lax.fori_loop(0, _FPC, field_body, 0)

    return k(x_t, tab_t)


_BS = 2048


def _scale_body(raw_ref, max_ref, out_ref):
    s = 0.2 / max_ref[0, 0, 0]
    e = raw_ref[...]
    z2 = (e * e) * (s * s)
    p = z2 * (-17.0 / 315.0) + (2.0 / 15.0)
    p = z2 * p + (-1.0 / 3.0)
    p = z2 * p + 1.0
    out_ref[...] = (e * s) * p


def _apply_tanh(raw, maxes):
    return pl.pallas_call(
        _scale_body,
        grid=(N_CAT, BATCH // _BS),
        in_specs=[
            pl.BlockSpec((EMB, _BS), lambda f, b: (f, b)),
            pl.BlockSpec((1, 1, 128), lambda f, b: (f, 0, 0)),
        ],
        out_specs=pl.BlockSpec((EMB, _BS), lambda f, b: (f, b)),
        out_shape=jax.ShapeDtypeStruct((RWS, BATCH), jnp.float32),
    )(raw, maxes)


def kernel(x, tables):
    tab_t = jnp.transpose(tables, (0, 2, 1)).reshape(RWS, VP1)
    x_t = jnp.transpose(x)
    maxes = _field_maxes(tab_t)
    raw = _sc_gather(x_t, tab_t)
    out_t = _apply_tanh(raw, maxes)
    return jnp.transpose(out_t.reshape(N_CAT, EMB, BATCH), (2, 0, 1))


# speedup vs baseline: 1.1790x; 1.1790x over previous
import functools

import jax
import jax.numpy as jnp
from jax import lax
from jax.experimental import pallas as pl
from jax.experimental.pallas import tpu as pltpu
from jax.experimental.pallas import tpu_sc as plsc

N_CAT = 26
VP1 = 100001
EMB = 32
BATCH = 16384
RWS = N_CAT * EMB

_NUM_CORES = 2
_NUM_SUBCORES = 16
_FPC = N_CAT // _NUM_CORES


_CH = 25088
_NCH = (VP1 + _CH - 1) // _CH


def _max_body(tab_ref, out_ref):
    k = pl.program_id(1)
    blk = jnp.abs(tab_ref[...])
    col = lax.broadcasted_iota(jnp.int32, blk.shape, 1) + k * _CH
    blk = jnp.where(col < VP1, blk, 0.0)
    m = jnp.max(blk)

    @pl.when(k == 0)
    def _():
        out_ref[0, 0, :] = jnp.full((128,), m)

    @pl.when(k > 0)
    def _():
        out_ref[0, 0, :] = jnp.maximum(out_ref[0, 0, :], m)


def _field_maxes(tab_t):
    return pl.pallas_call(
        _max_body,
        grid=(N_CAT, _NCH),
        in_specs=[pl.BlockSpec((EMB, _CH), lambda f, k: (f, k))],
        out_specs=pl.BlockSpec((1, 1, 128), lambda f, k: (f, 0, 0)),
        out_shape=jax.ShapeDtypeStruct((N_CAT, 1, 128), jnp.float32),
    )(tab_t)


_HB = 8192


def _sc_gather(x_t, tab_t):
    mesh = plsc.VectorSubcoreMesh(core_axis_name="c", subcore_axis_name="s")

    @functools.partial(
        pl.kernel,
        mesh=mesh,
        compiler_params=pltpu.CompilerParams(
            use_tc_tiling_on_sc=True, needs_layout_passes=False
        ),
        out_type=jax.ShapeDtypeStruct((RWS, BATCH), jnp.float32),
        scratch_types=[
            pltpu.VMEM((VP1,), jnp.float32),
            pltpu.VMEM((_HB,), jnp.int32),
            pltpu.VMEM((_HB,), jnp.float32),
        ],
    )
    def k(x_hbm, tab_hbm, raw_hbm, tvec_v, xcol_v, obuf_v):
        c = lax.axis_index("c")
        s = lax.axis_index("s")

        def field_body(tf, carry):
            f = c * _FPC + tf

            def j_body(jj, carry):
                r = f * EMB + s + _NUM_SUBCORES * jj
                pltpu.sync_copy(tab_hbm.at[r], tvec_v)

                def b_body(bc, carry):
                    b0 = bc * _HB
                    pltpu.sync_copy(x_hbm.at[f, pl.ds(b0, _HB)], xcol_v)

                    def v_body(v, carry):
                        iv = xcol_v[pl.ds(v * 16, 16)] + 1
                        obuf_v[pl.ds(v * 16, 16)] = plsc.load_gather(
                            tvec_v, [iv]
                        )
                        return carry

                    lax.fori_loop(0, _HB // 16, v_body, 0)
                    pltpu.sync_copy(obuf_v, raw_hbm.at[r, pl.ds(b0, _HB)])
                    return carry

                return lax.fori_loop(0, BATCH // _HB, b_body, carry)

            return lax.fori_loop(0, 2, j_body, carry)

        lax.fori_loop(0, _FPC, field_body, 0)

    return k(x_t, tab_t)


_BS = 2048


def _scale_body(raw_ref, max_ref, out_ref):
    s = 0.2 / max_ref[0, 0, 0]
    e = raw_ref[...]
    z2 = (e * e) * (s * s)
    p = z2 * (-17.0 / 315.0) + (2.0 / 15.0)
    p = z2 * p + (-1.0 / 3.0)
    p = z2 * p + 1.0
    out_ref[...] = (e * s) * p


def _apply_tanh(raw, maxes):
    return pl.pallas_call(
        _scale_body,
        grid=(N_CAT,),
        in_specs=[
            pl.BlockSpec((EMB, BATCH), lambda f: (f, 0)),
            pl.BlockSpec((1, 1, 128), lambda f: (f, 0, 0)),
        ],
        out_specs=pl.BlockSpec((EMB, BATCH), lambda f: (f, 0)),
        out_shape=jax.ShapeDtypeStruct((RWS, BATCH), jnp.float32),
    )(raw, maxes)


def kernel(x, tables):
    tab_t = jnp.transpose(tables, (0, 2, 1)).reshape(RWS, VP1)
    x_t = jnp.transpose(x)
    maxes = _field_maxes(tab_t)
    raw = _sc_gather(x_t, tab_t)
    out_t = _apply_tanh(raw, maxes)
    return jnp.transpose(out_t.reshape(N_CAT, EMB, BATCH), (2, 0, 1))
